# baseline (speedup 1.0000x reference)
import functools

import jax
import jax.numpy as jnp
from jax.experimental import pallas as pl
from jax.experimental.pallas import tpu as pltpu

NUM_PAGES, PAGE_SIZE, KV_DIM = 2048, 16, 1024
MAX_SEQS, MAX_LEN = 128, 8192
PAGES_PER_SEQ = MAX_LEN // PAGE_SIZE
B = 64


def _copy_body(sel_ref, in_ref, out_ref):
    out_ref[...] = in_ref[...]


def _gather_rows(x, sel, block_rows=1):
    n = x.shape[0]
    grid = (n // block_rows,)
    blk = (block_rows,) + x.shape[1:]
    return pl.pallas_call(
        _copy_body,
        grid_spec=pltpu.PrefetchScalarGridSpec(
            num_scalar_prefetch=1,
            grid=grid,
            in_specs=[pl.BlockSpec(blk, lambda i, sel_ref: (sel_ref[i], 0, 0))],
            out_specs=pl.BlockSpec(blk, lambda i, sel_ref: (i, 0, 0)),
        ),
        out_shape=jax.ShapeDtypeStruct(x.shape, x.dtype),
    )(sel, x)


def kernel(cache, tokens, kv_pages, seq_lens, parent_ids, child_ids, page_src, page_dst):
    sel = jnp.arange(NUM_PAGES, dtype=jnp.int32).at[page_dst].set(page_src)
    tsel = jnp.arange(MAX_SEQS, dtype=jnp.int32).at[child_ids].set(parent_ids)

    new_cache = _gather_rows(cache, sel)
    new_tokens = _gather_rows(tokens.reshape(MAX_SEQS, 1, MAX_LEN), tsel).reshape(MAX_SEQS, MAX_LEN)
    new_kv_pages = _gather_rows(
        kv_pages.reshape(MAX_SEQS, 1, PAGES_PER_SEQ), tsel
    ).reshape(MAX_SEQS, PAGES_PER_SEQ)
    new_seq_lens = seq_lens[tsel]
    return new_cache, new_tokens, new_kv_pages, new_seq_lens


# speedup vs baseline: 4.1380x; 4.1380x over previous
import functools

import jax
import jax.numpy as jnp
from jax.experimental import pallas as pl
from jax.experimental.pallas import tpu as pltpu

NUM_PAGES, PAGE_SIZE, KV_DIM = 2048, 16, 1024
MAX_SEQS, MAX_LEN = 128, 8192
PAGES_PER_SEQ = MAX_LEN // PAGE_SIZE
B = 64


def _copy_body(in_ref, out_ref):
    out_ref[...] = in_ref[...]


def _stream_copy(x, block_rows):
    n = x.shape[0]
    blk = (block_rows,) + x.shape[1:]
    ix = lambda i: (i,) + (0,) * (x.ndim - 1)
    return pl.pallas_call(
        _copy_body,
        grid=(n // block_rows,),
        in_specs=[pl.BlockSpec(blk, ix)],
        out_specs=pl.BlockSpec(blk, ix),
        out_shape=jax.ShapeDtypeStruct(x.shape, x.dtype),
    )(x)


def _fixup_body(cache_cp, tokens_cp, kv_cp, seqlens_ref, cache, tokens, kv_pages,
                parent_ref, child_ref, psrc_ref, pdst_ref,
                out_cache, out_tokens, out_kv, out_seqlens,
                sem_c, sem_t, sem_k):
    def cache_dma(i):
        return pltpu.make_async_copy(
            cache.at[psrc_ref[i]], out_cache.at[pdst_ref[i]], sem_c)

    def tok_dma(i):
        return pltpu.make_async_copy(
            tokens.at[parent_ref[i]], out_tokens.at[child_ref[i]], sem_t)

    def kv_dma(i):
        return pltpu.make_async_copy(
            kv_pages.at[parent_ref[i]], out_kv.at[child_ref[i]], sem_k)

    def issue(i, _):
        cache_dma(i).start()
        tok_dma(i).start()
        kv_dma(i).start()
        return 0

    jax.lax.fori_loop(0, B, issue, 0)

    def cp(i, _):
        out_seqlens[i] = seqlens_ref[i]
        return 0

    jax.lax.fori_loop(0, MAX_SEQS, cp, 0)

    def fix(i, _):
        out_seqlens[child_ref[i]] = seqlens_ref[parent_ref[i]]
        return 0

    jax.lax.fori_loop(0, B, fix, 0)

    def drain(i, _):
        cache_dma(i).wait()
        tok_dma(i).wait()
        kv_dma(i).wait()
        return 0

    jax.lax.fori_loop(0, B, drain, 0)


def kernel(cache, tokens, kv_pages, seq_lens, parent_ids, child_ids, page_src, page_dst):
    cache_cp = _stream_copy(cache, 128)
    tokens_cp = _stream_copy(tokens, 32)
    kv_cp = _stream_copy(kv_pages, 128)

    smem = functools.partial(pl.BlockSpec, memory_space=pltpu.SMEM)
    any_ = functools.partial(pl.BlockSpec, memory_space=pl.ANY)
    new_cache, new_tokens, new_kv, new_seqlens = pl.pallas_call(
        _fixup_body,
        in_specs=[any_(), any_(), any_(), smem(),
                  any_(), any_(), any_(),
                  smem(), smem(), smem(), smem()],
        out_specs=(any_(), any_(), any_(), smem()),
        out_shape=(
            jax.ShapeDtypeStruct(cache.shape, cache.dtype),
            jax.ShapeDtypeStruct(tokens.shape, tokens.dtype),
            jax.ShapeDtypeStruct(kv_pages.shape, kv_pages.dtype),
            jax.ShapeDtypeStruct(seq_lens.shape, seq_lens.dtype),
        ),
        input_output_aliases={0: 0, 1: 1, 2: 2},
        scratch_shapes=[pltpu.SemaphoreType.DMA] * 3,
    )(cache_cp, tokens_cp, kv_cp, seq_lens, cache, tokens, kv_pages,
      parent_ids, child_ids, page_src, page_dst)
    return new_cache, new_tokens, new_kv, new_seqlens
